# baseline (speedup 1.0000x reference)
import dataclasses
import functools

import jax
import jax.numpy as jnp
from jax import lax
from jax.experimental import pallas as pl
from jax.experimental.pallas import tpu as pltpu
from jax.experimental.pallas import tpu_sc as plsc

BINS = 2048
HMIN = -4.0
INV_W = 256.0

NC = 2
NS = 16
L = 16
NW = NC * NS

BLOCK = 16384

STRIDE = 2064


def _sc_partial_hists(x):
    n = x.shape[0]
    nblk = n // BLOCK
    mesh = plsc.VectorSubcoreMesh(core_axis_name="core", subcore_axis_name="subcore")
    cp = pltpu.CompilerParams()
    if "needs_layout_passes" in pltpu.CompilerParams.__dataclass_fields__:
        cp = dataclasses.replace(cp, needs_layout_passes=False)

    @functools.partial(
        pl.kernel,
        out_type=jax.ShapeDtypeStruct((NW * STRIDE * L,), jnp.float32),
        mesh=mesh,
        compiler_params=cp,
        scratch_types=[
            pltpu.VMEM((STRIDE * L,), jnp.float32),
        ],
    )
    def k(x_hbm, out_hbm, hist):
        wid = lax.axis_index("core") * NS + lax.axis_index("subcore")

        @pl.loop(0, STRIDE * L, step=8 * L)
        def _zero(i):
            for j in range(8):
                hist[pl.ds(i + j * L, L)] = jnp.zeros((L,), jnp.float32)

        lane = lax.iota(jnp.int32, L)
        ones = jnp.ones((L,), jnp.float32)

        def body(x_vmem):
            @plsc.parallel_loop(0, BLOCK, L, unroll=8)
            def _(c):
                v = x_vmem[pl.ds(c, L)]
                u = (v - HMIN) * INV_W
                idx = u.astype(jnp.int32)
                mask = jnp.abs(v) <= 4.0
                addr = lax.shift_left(idx, 4) + lane
                plsc.addupdate_scatter(hist, [addr], ones, mask=mask)

        pltpu.emit_pipeline(
            body,
            grid=(nblk,),
            in_specs=[pl.BlockSpec((BLOCK,), lambda i: (i,))],
            core_axis_name=("core", "subcore"),
            dimension_semantics=(pltpu.PARALLEL,),
        )(x_hbm)

        pltpu.sync_copy(hist, out_hbm.at[pl.ds(wid * STRIDE * L, STRIDE * L)])

    return k(x)


def _tc_combine(parts):
    def body(p_ref, o_ref):
        s = jnp.sum(p_ref[...].reshape(NW, STRIDE, L), axis=(0, 2))
        h = s[:BINS] + jnp.where(
            lax.iota(jnp.int32, BINS) == BINS - 1, s[BINS], 0.0
        )
        o_ref[...] = h.reshape(1, BINS)

    out = pl.pallas_call(
        body,
        out_shape=jax.ShapeDtypeStruct((1, BINS), jnp.float32),
    )(parts)
    return out.reshape(BINS)


def kernel(x, out):
    del out
    parts = _sc_partial_hists(x).reshape(NW * STRIDE, L)
    hist = _tc_combine(parts)
    return (hist, hist)


# speedup vs baseline: 1.1114x; 1.1114x over previous
import dataclasses
import functools

import jax
import jax.numpy as jnp
from jax import lax
from jax.experimental import pallas as pl
from jax.experimental.pallas import tpu as pltpu
from jax.experimental.pallas import tpu_sc as plsc

BINS = 2048
HMIN = -4.0
INV_W = 256.0

NC = 2
NS = 16
L = 16
NW = NC * NS

BLOCK = 16384

STRIDE = 2064


def _sc_partial_hists(x):
    n = x.shape[0]
    nblk = n // BLOCK
    mesh = plsc.VectorSubcoreMesh(core_axis_name="core", subcore_axis_name="subcore")
    cp = pltpu.CompilerParams()
    if "needs_layout_passes" in pltpu.CompilerParams.__dataclass_fields__:
        cp = dataclasses.replace(cp, needs_layout_passes=False)

    @functools.partial(
        pl.kernel,
        out_type=jax.ShapeDtypeStruct((NW, BINS), jnp.float32),
        mesh=mesh,
        compiler_params=cp,
        scratch_types=[
            pltpu.VMEM((STRIDE,), jnp.float32),
        ],
    )
    def k(x_hbm, out_hbm, hist):
        wid = lax.axis_index("core") * NS + lax.axis_index("subcore")

        @pl.loop(0, STRIDE, step=8 * L)
        def _zero(i):
            for j in range(8):
                hist[pl.ds(i + j * L, L)] = jnp.zeros((L,), jnp.float32)

        ones = jnp.ones((L,), jnp.float32)

        def body(x_vmem):
            @plsc.parallel_loop(0, BLOCK, L, unroll=8)
            def _(c):
                v = x_vmem[pl.ds(c, L)]
                u = (v - HMIN) * INV_W
                idx = u.astype(jnp.int32)
                mask = jnp.abs(v) <= 4.0
                plsc.addupdate_scatter(hist, [idx], ones, mask=mask)

        pltpu.emit_pipeline(
            body,
            grid=(nblk,),
            in_specs=[pl.BlockSpec((BLOCK,), lambda i: (i,))],
            core_axis_name=("core", "subcore"),
            dimension_semantics=(pltpu.PARALLEL,),
            trace_scopes=False,
        )(x_hbm)

        tail = hist[pl.ds(BINS - L, L)]
        extra = hist[pl.ds(BINS, L)]
        hist[pl.ds(BINS - L, L)] = tail + lax.rev(extra, (0,))

        pltpu.sync_copy(hist.at[pl.ds(0, BINS)], out_hbm.at[wid])

    return k(x)


def _tc_combine(parts):
    def body(p_ref, o_ref):
        o_ref[...] = jnp.sum(p_ref[...], axis=0, keepdims=True)

    out = pl.pallas_call(
        body,
        out_shape=jax.ShapeDtypeStruct((1, BINS), jnp.float32),
    )(parts)
    return out.reshape(BINS)


def kernel(x, out):
    del out
    parts = _sc_partial_hists(x)
    hist = _tc_combine(parts)
    return (hist, hist)
